# baseline (speedup 1.0000x reference)
import jax
import jax.numpy as jnp
from jax.experimental import pallas as pl
from jax.experimental.pallas import tpu as pltpu

_B, _D, _C = 256, 512, 100000
_UM, _LM = 0.8, 0.45
_UA, _LA = 110.0, 10.0
_LAMBDA_G = 35.0
_SCALE = 64.0

_CB = 2000
_NBLK = _C // _CB
_NEG = -1e30


def _body(xt_ref, xn1_ref, lab_ref, w_ref, o_ref,
          xhi_scr, xcat_scr, m_scr, s_scr, t_scr):
    j = pl.program_id(0)

    @pl.when(j == 0)
    def _init():
        xt = xt_ref[...]
        rx = jax.lax.rsqrt(jnp.sum(xt * xt, axis=0, keepdims=True))
        xn = xt * rx
        xhi = xn.astype(jnp.bfloat16)
        xlo = (xn - xhi.astype(jnp.float32)).astype(jnp.bfloat16)
        xhi_scr[...] = xhi
        xcat_scr[:_D] = xlo
        xcat_scr[_D:] = xhi
        m_scr[...] = jnp.full_like(m_scr, _NEG)
        s_scr[...] = jnp.zeros_like(s_scr)
        t_scr[...] = jnp.zeros_like(t_scr)

    w = w_ref[...]
    rws = _SCALE * jax.lax.rsqrt(jnp.sum(w * w, axis=1, keepdims=True))
    whi = w.astype(jnp.bfloat16)
    wlo = (w - whi.astype(jnp.float32)).astype(jnp.bfloat16)
    wcat = jnp.concatenate([whi, wlo], axis=1)
    dn = (((1,), (0,)), ((), ()))
    raw = (jax.lax.dot_general(whi, xhi_scr[...], dn,
                               preferred_element_type=jnp.float32)
           + jax.lax.dot_general(wcat, xcat_scr[...], dn,
                                 preferred_element_type=jnp.float32))
    logits = raw * rws

    rows = j * _CB + jax.lax.broadcasted_iota(jnp.int32, (_CB, _B), 0)
    hit = rows == lab_ref[...]

    bm = jnp.max(logits, axis=0, keepdims=True)
    m_new = jnp.maximum(m_scr[...], bm)
    p = jnp.exp(logits - m_new)
    s_scr[...] = (s_scr[...] * jnp.exp(m_scr[...] - m_new)
                  + jnp.sum(p, axis=0, keepdims=True))
    m_scr[...] = m_new
    t_scr[...] = t_scr[...] + jnp.sum(jnp.where(hit, logits, 0.0),
                                      axis=0, keepdims=True)

    @pl.when(j == _NBLK - 1)
    def _fin():
        m = m_scr[...]
        s0 = t_scr[...]
        cosl = jnp.clip(s0 * (1.0 / _SCALE), -1.0, 1.0)
        a = xn1_ref[...]
        ada = (_UM - _LM) / (_UA - _LA) * (a - _LA) + _LM
        ctm = cosl * jnp.cos(ada) - jnp.sqrt(1.0 - cosl * cosl) * jnp.sin(ada)
        ctm = jnp.where(cosl > 0.0, ctm, cosl)
        sm = _SCALE * ctm
        s_new = (jnp.maximum(s_scr[...] - jnp.exp(s0 - m), 0.0)
                 + jnp.exp(sm - m))
        lse = jnp.log(s_new) + m
        g = a * (1.0 / (_UA * _UA)) + 1.0 / a
        tot = jnp.sum((lse - sm) + _LAMBDA_G * g, axis=1, keepdims=True)
        o_ref[...] = tot * (1.0 / _B)


def kernel(x, x_norm, labels, weight):
    xt = x.T
    xn1 = x_norm.reshape(1, _B)
    lab1 = labels.reshape(1, _B).astype(jnp.int32)

    out = pl.pallas_call(
        _body,
        grid=(_NBLK,),
        in_specs=[
            pl.BlockSpec((_D, _B), lambda j: (0, 0)),
            pl.BlockSpec((1, _B), lambda j: (0, 0)),
            pl.BlockSpec((1, _B), lambda j: (0, 0)),
            pl.BlockSpec((_CB, _D), lambda j: (j, 0)),
        ],
        out_specs=pl.BlockSpec((1, 1), lambda j: (0, 0)),
        out_shape=jax.ShapeDtypeStruct((1, 1), jnp.float32),
        scratch_shapes=[
            pltpu.VMEM((_D, _B), jnp.bfloat16),
            pltpu.VMEM((2 * _D, _B), jnp.bfloat16),
            pltpu.VMEM((1, _B), jnp.float32),
            pltpu.VMEM((1, _B), jnp.float32),
            pltpu.VMEM((1, _B), jnp.float32),
        ],
        compiler_params=pltpu.CompilerParams(
            dimension_semantics=("arbitrary",),
            vmem_limit_bytes=56 * 1024 * 1024,
        ),
        name="magface_loss",
    )(xt, xn1, lab1, weight)
    return out[0, 0]


# speedup vs baseline: 1.3616x; 1.3616x over previous
import jax
import jax.numpy as jnp
from jax.experimental import pallas as pl
from jax.experimental.pallas import tpu as pltpu

_B, _D, _C = 256, 512, 100000
_UM, _LM = 0.8, 0.45
_UA, _LA = 110.0, 10.0
_LAMBDA_G = 35.0
_SCALE = 64.0

_CB = 2000
_NBLK = _C // _CB
_NEG = -1e30


def _body(xt_ref, x_ref, xn1_ref, lab_ref, labs_ref, w_ref, w_any_ref, o_ref,
          xhi_scr, grow_scr, m_scr, s_scr, t_scr, gsem):
    j = pl.program_id(0)

    @pl.when(j == 0)
    def _init():
        xt = xt_ref[...]
        rx = jax.lax.rsqrt(jnp.sum(xt * xt, axis=0, keepdims=True))
        xhi_scr[...] = (xt * rx).astype(jnp.bfloat16)
        m_scr[...] = jnp.full_like(m_scr, _NEG)
        s_scr[...] = jnp.zeros_like(s_scr)
        t_scr[...] = jnp.zeros_like(t_scr)

        def _issue(b, _):
            lab = labs_ref[0, b]
            pltpu.make_async_copy(w_any_ref.at[pl.ds(lab, 1), :],
                                  grow_scr.at[pl.ds(b, 1), :],
                                  gsem).start()
            return 0
        jax.lax.fori_loop(0, _B, _issue, 0)

    w = w_ref[...]
    rws = _SCALE * jax.lax.rsqrt(jnp.sum(w * w, axis=1, keepdims=True))
    raw = jax.lax.dot_general(
        w.astype(jnp.bfloat16), xhi_scr[...], (((1,), (0,)), ((), ())),
        preferred_element_type=jnp.float32)
    logits = raw * rws

    rows = j * _CB + jax.lax.broadcasted_iota(jnp.int32, (_CB, _B), 0)
    hit = rows == lab_ref[...]

    bm = jnp.max(logits, axis=0, keepdims=True)
    m_new = jnp.maximum(m_scr[...], bm)
    p = jnp.exp(logits - m_new)
    s_scr[...] = (s_scr[...] * jnp.exp(m_scr[...] - m_new)
                  + jnp.sum(p, axis=0, keepdims=True))
    m_scr[...] = m_new
    t_scr[...] = t_scr[...] + jnp.sum(jnp.where(hit, logits, 0.0),
                                      axis=0, keepdims=True)

    @pl.when(j == _NBLK - 1)
    def _fin():
        def _drain(b, _):
            pltpu.make_async_copy(w_any_ref.at[pl.ds(0, 1), :],
                                  grow_scr.at[pl.ds(0, 1), :],
                                  gsem).wait()
            return 0
        jax.lax.fori_loop(0, _B, _drain, 0)

        xb = x_ref[...]
        gw = grow_scr[...]
        dotv = jnp.sum(gw * xb, axis=1, keepdims=True)
        nw = jnp.sum(gw * gw, axis=1, keepdims=True)
        nx = jnp.sum(xb * xb, axis=1, keepdims=True)
        cos_col = dotv * jax.lax.rsqrt(nw * nx)
        eye = (jax.lax.broadcasted_iota(jnp.int32, (_B, _B), 0)
               == jax.lax.broadcasted_iota(jnp.int32, (_B, _B), 1)
               ).astype(jnp.float32)
        cos_row = jax.lax.dot_general(
            cos_col, eye, (((0,), (0,)), ((), ())),
            preferred_element_type=jnp.float32,
            precision=jax.lax.Precision.HIGHEST)

        m = m_scr[...]
        s0 = t_scr[...]
        cosl = jnp.clip(cos_row, -1.0, 1.0)
        a = xn1_ref[...]
        ada = (_UM - _LM) / (_UA - _LA) * (a - _LA) + _LM
        ctm = cosl * jnp.cos(ada) - jnp.sqrt(1.0 - cosl * cosl) * jnp.sin(ada)
        ctm = jnp.where(cosl > 0.0, ctm, cosl)
        sm = _SCALE * ctm
        s_new = (jnp.maximum(s_scr[...] - jnp.exp(s0 - m), 0.0)
                 + jnp.exp(sm - m))
        lse = jnp.log(s_new) + m
        g = a * (1.0 / (_UA * _UA)) + 1.0 / a
        tot = jnp.sum((lse - sm) + _LAMBDA_G * g, axis=1, keepdims=True)
        o_ref[...] = tot * (1.0 / _B)


def kernel(x, x_norm, labels, weight):
    xt = x.T
    xn1 = x_norm.reshape(1, _B)
    lab1 = labels.reshape(1, _B).astype(jnp.int32)

    out = pl.pallas_call(
        _body,
        grid=(_NBLK,),
        in_specs=[
            pl.BlockSpec((_D, _B), lambda j: (0, 0)),
            pl.BlockSpec((_B, _D), lambda j: (0, 0)),
            pl.BlockSpec((1, _B), lambda j: (0, 0)),
            pl.BlockSpec((1, _B), lambda j: (0, 0)),
            pl.BlockSpec(memory_space=pltpu.SMEM),
            pl.BlockSpec((_CB, _D), lambda j: (j, 0)),
            pl.BlockSpec(memory_space=pl.ANY),
        ],
        out_specs=pl.BlockSpec((1, 1), lambda j: (0, 0)),
        out_shape=jax.ShapeDtypeStruct((1, 1), jnp.float32),
        scratch_shapes=[
            pltpu.VMEM((_D, _B), jnp.bfloat16),
            pltpu.VMEM((_B, _D), jnp.float32),
            pltpu.VMEM((1, _B), jnp.float32),
            pltpu.VMEM((1, _B), jnp.float32),
            pltpu.VMEM((1, _B), jnp.float32),
            pltpu.SemaphoreType.DMA,
        ],
        compiler_params=pltpu.CompilerParams(
            dimension_semantics=("arbitrary",),
            vmem_limit_bytes=56 * 1024 * 1024,
        ),
        name="magface_loss",
    )(xt, x, xn1, lab1, lab1, weight, weight)
    return out[0, 0]


# speedup vs baseline: 1.4414x; 1.0586x over previous
import jax
import jax.numpy as jnp
from jax.experimental import pallas as pl
from jax.experimental.pallas import tpu as pltpu

_B, _D, _C = 256, 512, 100000
_UM, _LM = 0.8, 0.45
_UA, _LA = 110.0, 10.0
_LAMBDA_G = 35.0
_SCALE = 64.0
_LOG2E = 1.4426950408889634
_LN2 = 0.6931471805599453

_CB = 2000
_NBLK = _C // _CB
_NEG = -1e30


def _body(xt_ref, x_ref, xn1_ref, labs_ref, w_ref, w_any_ref, o_ref,
          xhi_scr, grow_scr, m_scr, s_scr, gsem):
    j = pl.program_id(0)

    @pl.when(j == 0)
    def _init():
        xt = xt_ref[...]
        rx = jax.lax.rsqrt(jnp.sum(xt * xt, axis=0, keepdims=True))
        xhi_scr[...] = (xt * rx).astype(jnp.bfloat16)
        m_scr[...] = jnp.full_like(m_scr, _NEG)
        s_scr[...] = jnp.zeros_like(s_scr)

        def _issue(b, _):
            lab = labs_ref[0, b]
            pltpu.make_async_copy(w_any_ref.at[pl.ds(lab, 1), :],
                                  grow_scr.at[pl.ds(b, 1), :],
                                  gsem).start()
            return 0
        jax.lax.fori_loop(0, _B, _issue, 0)

    w = w_ref[...]
    rws2 = ((_SCALE * _LOG2E)
            * jax.lax.rsqrt(jnp.sum(w * w, axis=1, keepdims=True)))
    raw = jax.lax.dot_general(
        w.astype(jnp.bfloat16), xhi_scr[...], (((1,), (0,)), ((), ())),
        preferred_element_type=jnp.float32)
    arg2 = raw * rws2

    bm = jnp.max(arg2, axis=0, keepdims=True)
    m_new = jnp.maximum(m_scr[...], bm)
    p = jnp.exp2(arg2 - m_new)
    s_scr[...] = (s_scr[...] * jnp.exp2(m_scr[...] - m_new)
                  + jnp.sum(p, axis=0, keepdims=True))
    m_scr[...] = m_new

    @pl.when(j == _NBLK - 1)
    def _fin():
        def _drain(b, _):
            pltpu.make_async_copy(w_any_ref.at[pl.ds(0, 1), :],
                                  grow_scr.at[pl.ds(0, 1), :],
                                  gsem).wait()
            return 0
        jax.lax.fori_loop(0, _B, _drain, 0)

        xb = x_ref[...]
        gw = grow_scr[...]
        dotv = jnp.sum(gw * xb, axis=1, keepdims=True)
        nw = jnp.sum(gw * gw, axis=1, keepdims=True)
        nx = jnp.sum(xb * xb, axis=1, keepdims=True)
        cos_col = dotv * jax.lax.rsqrt(nw * nx)
        eye = (jax.lax.broadcasted_iota(jnp.int32, (_B, _B), 0)
               == jax.lax.broadcasted_iota(jnp.int32, (_B, _B), 1)
               ).astype(jnp.float32)
        cos_row = jax.lax.dot_general(
            cos_col, eye, (((0,), (0,)), ((), ())),
            preferred_element_type=jnp.float32,
            precision=jax.lax.Precision.HIGHEST)

        m2 = m_scr[...]
        cosl = jnp.clip(cos_row, -1.0, 1.0)
        a = xn1_ref[...]
        ada = (_UM - _LM) / (_UA - _LA) * (a - _LA) + _LM
        ctm = cosl * jnp.cos(ada) - jnp.sqrt(1.0 - cosl * cosl) * jnp.sin(ada)
        ctm = jnp.where(cosl > 0.0, ctm, cosl)
        s0_2 = (_SCALE * _LOG2E) * cosl
        sm_2 = (_SCALE * _LOG2E) * ctm
        s_new = (jnp.maximum(s_scr[...] - jnp.exp2(s0_2 - m2), 0.0)
                 + jnp.exp2(sm_2 - m2))
        lse = (jnp.log2(s_new) + m2) * _LN2
        g = a * (1.0 / (_UA * _UA)) + 1.0 / a
        tot = jnp.sum((lse - sm_2 * _LN2) + _LAMBDA_G * g,
                      axis=1, keepdims=True)
        o_ref[...] = tot * (1.0 / _B)


def kernel(x, x_norm, labels, weight):
    xt = x.T
    xn1 = x_norm.reshape(1, _B)
    lab1 = labels.reshape(1, _B).astype(jnp.int32)

    out = pl.pallas_call(
        _body,
        grid=(_NBLK,),
        in_specs=[
            pl.BlockSpec((_D, _B), lambda j: (0, 0)),
            pl.BlockSpec((_B, _D), lambda j: (0, 0)),
            pl.BlockSpec((1, _B), lambda j: (0, 0)),
            pl.BlockSpec(memory_space=pltpu.SMEM),
            pl.BlockSpec((_CB, _D), lambda j: (j, 0)),
            pl.BlockSpec(memory_space=pl.ANY),
        ],
        out_specs=pl.BlockSpec((1, 1), lambda j: (0, 0)),
        out_shape=jax.ShapeDtypeStruct((1, 1), jnp.float32),
        scratch_shapes=[
            pltpu.VMEM((_D, _B), jnp.bfloat16),
            pltpu.VMEM((_B, _D), jnp.float32),
            pltpu.VMEM((1, _B), jnp.float32),
            pltpu.VMEM((1, _B), jnp.float32),
            pltpu.SemaphoreType.DMA,
        ],
        compiler_params=pltpu.CompilerParams(
            dimension_semantics=("arbitrary",),
            vmem_limit_bytes=56 * 1024 * 1024,
        ),
        name="magface_loss",
    )(xt, x, xn1, lab1, weight, weight)
    return out[0, 0]
